# baseline (speedup 1.0000x reference)
import functools

import jax
import jax.numpy as jnp
from jax import lax
from jax.experimental import pallas as pl
from jax.experimental.pallas import tpu as pltpu
from jax.experimental.pallas import tpu_sc as plsc

F32 = jnp.float32
I32 = jnp.int32

_TILES = 16
_CORES = 2
_CH = 80
_G = _CH // 16
_WD = 144


def _make_prep(N, IN, OUT, H, R, BN):
    nj = N // BN

    def body(x_ref, w_ref, we_ref, rel_ref, asrc_ref, adst_ref, aedge_ref,
             ht_ref, tdst_ref, relt_ref):
        j = pl.program_id(0)
        hb = jnp.dot(x_ref[...], w_ref[...], preferred_element_type=F32)
        dcols = []
        for h in range(H):
            hh = hb[:, h * OUT:(h + 1) * OUT]
            sc = jnp.sum(hh * asrc_ref[h][None, :], axis=1)[:, None]
            pad = jnp.zeros((hh.shape[0], _WD - OUT - 1), F32)
            ht_ref[h] = jnp.concatenate([hh, sc, pad], axis=1)
            dcols.append(jnp.sum(hh * adst_ref[h][None, :], axis=1)[:, None])
        zpad = jnp.zeros((dcols[0].shape[0], 16 - H), F32)
        tdst_ref[...] = jnp.concatenate(dcols + [zpad], axis=1)

        @pl.when(j == 0)
        def _():
            her = jnp.dot(rel_ref[...], we_ref[...],
                          preferred_element_type=F32)
            rcols = [jnp.sum(her[:, h * OUT:(h + 1) * OUT] *
                             aedge_ref[h][None, :], axis=1)[:, None]
                     for h in range(H)]
            relt_ref[...] = jnp.concatenate(rcols, axis=1)

    return pl.pallas_call(
        body,
        grid=(nj,),
        in_specs=[
            pl.BlockSpec((BN, IN), lambda j: (j, 0)),
            pl.BlockSpec((IN, H * OUT), lambda j: (0, 0)),
            pl.BlockSpec((IN, H * OUT), lambda j: (0, 0)),
            pl.BlockSpec((R, IN), lambda j: (0, 0)),
            pl.BlockSpec((H, OUT), lambda j: (0, 0)),
            pl.BlockSpec((H, OUT), lambda j: (0, 0)),
            pl.BlockSpec((H, OUT), lambda j: (0, 0)),
        ],
        out_specs=[
            pl.BlockSpec((H, BN, _WD), lambda j: (0, j, 0)),
            pl.BlockSpec((BN, 16), lambda j: (j, 0)),
            pl.BlockSpec((R, H), lambda j: (0, 0)),
        ],
        out_shape=[
            jax.ShapeDtypeStruct((H, N, _WD), F32),
            jax.ShapeDtypeStruct((N, 16), F32),
            jax.ShapeDtypeStruct((R, H), F32),
        ],
    )


def _make_sc(N, NP, OUT, H, R, E):
    EC = E // _TILES
    NCH = EC // _CH
    SR = NP // _TILES
    ZR = 16
    HC = H // _CORES
    mesh = plsc.VectorSubcoreMesh(core_axis_name="c", subcore_axis_name="s")

    @functools.partial(
        pl.kernel,
        out_type=jax.ShapeDtypeStruct((H * NP, _WD), F32),
        mesh=mesh,
        compiler_params=pltpu.CompilerParams(needs_layout_passes=False,
                                             use_tc_tiling_on_sc=False),
        scratch_types=[
            pltpu.VMEM((R * H,), F32),
            pltpu.VMEM((3, _CH), I32),
            pltpu.VMEM((3, _CH), I32),
            pltpu.VMEM((1, _CH), I32),
            pltpu.VMEM((1, _CH), I32),
            pltpu.VMEM((_CH,), F32),
            pltpu.VMEM((_CH,), F32),
            pltpu.VMEM((_CH, _WD), F32),
            pltpu.VMEM((_CH, _WD), F32),
            pltpu.VMEM((_CH, 16), F32),
            pltpu.VMEM((_CH, 16), F32),
            pltpu.VMEM((ZR, _WD), F32),
            pltpu.VMEM_SHARED((NP, _WD), F32),
            pltpu.SemaphoreType.DMA,
            pltpu.SemaphoreType.DMA,
            pltpu.SemaphoreType.DMA,
            pltpu.SemaphoreType.DMA,
            pltpu.SemaphoreType.DMA,
            pltpu.SemaphoreType.DMA,
            pltpu.SemaphoreType.DMA,
            pltpu.SemaphoreType.DMA,
        ],
    )
    def sc_kernel(epack_hbm, ht_hbm, tdst_hbm, relt_hbm,
                  msum_hbm,
                  Rl, ebuf0, ebuf1, idxb0, idxb1, exb0, exb1,
                  rowbuf0, rowbuf1, dbuf0, dbuf1, zacc, acc_sh,
                  esem0, esem1, gsem0, gsem1, ssem0, ssem1, dsem0, dsem1):
        cid = lax.axis_index("c")
        sid = lax.axis_index("s")
        iota16 = lax.iota(I32, 16)

        def _zi(i, _):
            for q in range(_WD // 16):
                zacc[i, pl.ds(q * 16, 16)] = jnp.zeros((16,), F32)
            return 0
        lax.fori_loop(0, ZR, _zi, 0)

        pltpu.sync_copy(relt_hbm, Rl)

        for hp in range(HC):
            h = cid * HC + hp
            hN = h * N
            hNP = h * NP


            for t in range(SR // ZR):
                pltpu.sync_copy(zacc, acc_sh.at[pl.ds(sid * SR + t * ZR, ZR)])

            plsc.subcore_barrier()

            def _proc(ebuf, idxb, exb, rowbuf, dbuf):
                for g in range(_G):
                    ev = g * 16 + iota16
                    dv = ebuf[1, pl.ds(g * 16, 16)]
                    tv = ebuf[2, pl.ds(g * 16, 16)]
                    al = (plsc.load_gather(rowbuf,
                                           [ev, jnp.full((16,), OUT, I32)]) +
                          plsc.load_gather(dbuf, [ev, jnp.full((16,), 0, I32) + h]) +
                          plsc.load_gather(Rl, [tv * H + h]))
                    al = jnp.maximum(al, al * F32(0.2))
                    ex = jnp.exp(al)
                    exb[pl.ds(g * 16, 16)] = ex
                    plsc.store_scatter(rowbuf,
                                       [ev, jnp.full((16,), OUT, I32)], ex)

                def _mul(g, _):
                    ev = exb[pl.ds(g * 16, 16)]
                    for i in range(16):
                        bi = ev.at[jnp.full((16,), i, I32)].get(
                            mode="promise_in_bounds")
                        e = g * 16 + i
                        for q in range(OUT // 16):
                            rowbuf[e, pl.ds(q * 16, 16)] = (
                                rowbuf[e, pl.ds(q * 16, 16)] * bi)
                    return 0
                lax.fori_loop(0, _G, _mul, 0)

            def _mkidx(ebuf, idxb):
                for g in range(_G):
                    sv = ebuf[0, pl.ds(g * 16, 16)]
                    idxb[0, pl.ds(g * 16, 16)] = sv + hN

            tbase = sid * (EC // _CH)

            def _fetch(cid0, ebuf, idxb, rowbuf, dbuf, esem, gsem, dsem):
                pltpu.async_copy(epack_hbm.at[cid0], ebuf, esem).wait()
                _mkidx(ebuf, idxb)
                pltpu.async_copy(ht_hbm.at[idxb.at[0]], rowbuf, gsem)
                pltpu.async_copy(tdst_hbm.at[ebuf.at[1]], dbuf, dsem)

            _fetch(tbase, ebuf0, idxb0, rowbuf0, dbuf0, esem0, gsem0, dsem0)
            _fetch(tbase + 1, ebuf1, idxb1, rowbuf1, dbuf1, esem1, gsem1,
                   dsem1)

            def _pair(j, _):
                pltpu.make_async_copy(ht_hbm.at[idxb0.at[0]], rowbuf0,
                                      gsem0).wait()
                pltpu.make_async_copy(tdst_hbm.at[ebuf0.at[1]], dbuf0,
                                      dsem0).wait()
                _proc(ebuf0, idxb0, exb0, rowbuf0, dbuf0)
                s0 = pltpu.async_copy(rowbuf0, acc_sh.at[ebuf0.at[1]], ssem0,
                                      add=True)
                pltpu.make_async_copy(ht_hbm.at[idxb1.at[0]], rowbuf1,
                                      gsem1).wait()
                pltpu.make_async_copy(tdst_hbm.at[ebuf1.at[1]], dbuf1,
                                      dsem1).wait()
                _proc(ebuf1, idxb1, exb1, rowbuf1, dbuf1)
                s1 = pltpu.async_copy(rowbuf1, acc_sh.at[ebuf1.at[1]], ssem1,
                                      add=True)
                nc = jnp.minimum(2 * j + 2, NCH - 2)
                s0.wait()
                s1.wait()
                ne0 = pltpu.async_copy(epack_hbm.at[tbase + nc], ebuf0, esem0)
                ne1 = pltpu.async_copy(epack_hbm.at[tbase + nc + 1], ebuf1,
                                       esem1)
                ne0.wait()
                _mkidx(ebuf0, idxb0)
                pltpu.async_copy(ht_hbm.at[idxb0.at[0]], rowbuf0, gsem0)
                pltpu.async_copy(tdst_hbm.at[ebuf0.at[1]], dbuf0, dsem0)
                ne1.wait()
                _mkidx(ebuf1, idxb1)
                pltpu.async_copy(ht_hbm.at[idxb1.at[0]], rowbuf1, gsem1)
                pltpu.async_copy(tdst_hbm.at[ebuf1.at[1]], dbuf1, dsem1)
                return 0

            lax.fori_loop(0, NCH // 2, _pair, 0)

            pltpu.make_async_copy(ht_hbm.at[idxb0.at[0]], rowbuf0,
                                  gsem0).wait()
            pltpu.make_async_copy(tdst_hbm.at[ebuf0.at[1]], dbuf0,
                                  dsem0).wait()
            pltpu.make_async_copy(ht_hbm.at[idxb1.at[0]], rowbuf1,
                                  gsem1).wait()
            pltpu.make_async_copy(tdst_hbm.at[ebuf1.at[1]], dbuf1,
                                  dsem1).wait()

            plsc.subcore_barrier()

            pltpu.sync_copy(acc_sh.at[pl.ds(sid * SR, SR)],
                            msum_hbm.at[pl.ds(hNP + sid * SR, SR)])

    return sc_kernel


def _final_body(msum_ref, bias_ref, out_ref):
    m = msum_ref[..., 0:128]
    d = msum_ref[..., 128:129]
    s = jnp.sum(m / (d + F32(1e-16)), axis=0) * F32(0.25)
    out_ref[...] = s + bias_ref[...]


def _make_final(N, NP, OUT, H, BN):
    return pl.pallas_call(
        _final_body,
        grid=(N // BN,),
        in_specs=[
            pl.BlockSpec((H, BN, _WD), lambda j: (0, j, 0)),
            pl.BlockSpec((1, OUT), lambda j: (0, 0)),
        ],
        out_specs=pl.BlockSpec((BN, OUT), lambda j: (j, 0)),
        out_shape=jax.ShapeDtypeStruct((N, OUT), F32),
    )


def kernel(x, edge_index, edge_type, rel_emb, W, W_edge, att_src, att_dst,
           att_edge, bias):
    N, IN = x.shape
    H, OUT = att_src.shape
    R = rel_emb.shape[0]
    E = edge_type.shape[0]
    NP = (N + _TILES * 16 - 1) // (_TILES * 16) * (_TILES * 16)
    assert E % (_TILES * _CH) == 0 and (NP // _TILES) % 16 == 0

    epack = jnp.stack([edge_index[0].reshape(E // _CH, _CH),
                       edge_index[1].reshape(E // _CH, _CH),
                       edge_type.reshape(E // _CH, _CH)], axis=1)

    ht, tdst, relt = _make_prep(N, IN, OUT, H, R, 400)(
        x, W, W_edge, rel_emb, att_src, att_dst, att_edge)

    msum = _make_sc(N, NP, OUT, H, R, E)(
        epack, ht.reshape(H * N, _WD), tdst, relt.reshape(R * H))

    out = _make_final(N, NP, OUT, H, 400)(
        msum.reshape(H, NP, _WD), bias.reshape(1, OUT))
    return out


# speedup vs baseline: 1.1899x; 1.1899x over previous
import functools

import jax
import jax.numpy as jnp
from jax import lax
from jax.experimental import pallas as pl
from jax.experimental.pallas import tpu as pltpu
from jax.experimental.pallas import tpu_sc as plsc

F32 = jnp.float32
I32 = jnp.int32

_TILES = 16
_CORES = 2
_CH = 80
_G = _CH // 16
_WD = 144


def _make_prep(N, IN, OUT, H, R, BN):
    nj = N // BN

    def body(x_ref, w_ref, we_ref, rel_ref, asrc_ref, adst_ref, aedge_ref,
             ht_ref, tdst_ref, relt_ref):
        j = pl.program_id(0)
        hb = jnp.dot(x_ref[...], w_ref[...], preferred_element_type=F32)
        dcols = []
        for h in range(H):
            hh = hb[:, h * OUT:(h + 1) * OUT]
            sc = jnp.sum(hh * asrc_ref[h][None, :], axis=1)[:, None]
            pad = jnp.zeros((hh.shape[0], _WD - OUT - 1), F32)
            ht_ref[h] = jnp.concatenate([hh, sc, pad], axis=1)
            dcols.append(jnp.sum(hh * adst_ref[h][None, :], axis=1)[:, None])
        zpad = jnp.zeros((dcols[0].shape[0], 16 - H), F32)
        tdst_ref[...] = jnp.concatenate(dcols + [zpad], axis=1)

        @pl.when(j == 0)
        def _():
            her = jnp.dot(rel_ref[...], we_ref[...],
                          preferred_element_type=F32)
            rcols = [jnp.sum(her[:, h * OUT:(h + 1) * OUT] *
                             aedge_ref[h][None, :], axis=1)[:, None]
                     for h in range(H)]
            relt_ref[...] = jnp.concatenate(rcols, axis=1)

    return pl.pallas_call(
        body,
        grid=(nj,),
        in_specs=[
            pl.BlockSpec((BN, IN), lambda j: (j, 0)),
            pl.BlockSpec((IN, H * OUT), lambda j: (0, 0)),
            pl.BlockSpec((IN, H * OUT), lambda j: (0, 0)),
            pl.BlockSpec((R, IN), lambda j: (0, 0)),
            pl.BlockSpec((H, OUT), lambda j: (0, 0)),
            pl.BlockSpec((H, OUT), lambda j: (0, 0)),
            pl.BlockSpec((H, OUT), lambda j: (0, 0)),
        ],
        out_specs=[
            pl.BlockSpec((H, BN, _WD), lambda j: (0, j, 0)),
            pl.BlockSpec((BN, 16), lambda j: (j, 0)),
            pl.BlockSpec((R, H), lambda j: (0, 0)),
        ],
        out_shape=[
            jax.ShapeDtypeStruct((H, N, _WD), F32),
            jax.ShapeDtypeStruct((N, 16), F32),
            jax.ShapeDtypeStruct((R, H), F32),
        ],
    )


def _make_sc(N, NP, OUT, H, R, E):
    EC = E // _TILES
    NCH = EC // _CH
    SR = NP // _TILES
    ZR = 16
    HC = H // _CORES
    mesh = plsc.VectorSubcoreMesh(core_axis_name="c", subcore_axis_name="s")

    @functools.partial(
        pl.kernel,
        out_type=jax.ShapeDtypeStruct((H * NP, _WD), F32),
        mesh=mesh,
        compiler_params=pltpu.CompilerParams(needs_layout_passes=False,
                                             use_tc_tiling_on_sc=False),
        scratch_types=[
            pltpu.VMEM((R * H,), F32),
            pltpu.VMEM((3, _CH), I32),
            pltpu.VMEM((3, _CH), I32),
            pltpu.VMEM((1, _CH), I32),
            pltpu.VMEM((1, _CH), I32),
            pltpu.VMEM((_CH,), F32),
            pltpu.VMEM((_CH,), F32),
            pltpu.VMEM((_CH, _WD), F32),
            pltpu.VMEM((_CH, _WD), F32),
            pltpu.VMEM((_CH, 16), F32),
            pltpu.VMEM((_CH, 16), F32),
            pltpu.VMEM((ZR, _WD), F32),
            pltpu.VMEM_SHARED((NP, _WD), F32),
            pltpu.SemaphoreType.DMA,
            pltpu.SemaphoreType.DMA,
            pltpu.SemaphoreType.DMA,
            pltpu.SemaphoreType.DMA,
            pltpu.SemaphoreType.DMA,
            pltpu.SemaphoreType.DMA,
            pltpu.SemaphoreType.DMA,
            pltpu.SemaphoreType.DMA,
        ],
    )
    def sc_kernel(epack_hbm, ht_hbm, tdst_hbm, relt_hbm,
                  msum_hbm,
                  Rl, ebuf0, ebuf1, idxb0, idxb1, exb0, exb1,
                  rowbuf0, rowbuf1, dbuf0, dbuf1, zacc, acc_sh,
                  esem0, esem1, gsem0, gsem1, ssem0, ssem1, dsem0, dsem1):
        cid = lax.axis_index("c")
        sid = lax.axis_index("s")
        iota16 = lax.iota(I32, 16)

        def _zi(i, _):
            for q in range(_WD // 16):
                zacc[i, pl.ds(q * 16, 16)] = jnp.zeros((16,), F32)
            return 0
        lax.fori_loop(0, ZR, _zi, 0)

        pltpu.sync_copy(relt_hbm, Rl)

        for hp in range(HC):
            h = cid * HC + hp
            hN = h * N
            hNP = h * NP


            for t in range(SR // ZR):
                pltpu.sync_copy(zacc, acc_sh.at[pl.ds(sid * SR + t * ZR, ZR)])

            plsc.subcore_barrier()

            def _proc(ebuf, idxb, exb, rowbuf, dbuf):
                for g in range(_G):
                    ev = g * 16 + iota16
                    dv = ebuf[1, pl.ds(g * 16, 16)]
                    tv = ebuf[2, pl.ds(g * 16, 16)]
                    al = (plsc.load_gather(rowbuf,
                                           [ev, jnp.full((16,), OUT, I32)]) +
                          plsc.load_gather(dbuf, [ev, jnp.full((16,), 0, I32) + h]) +
                          plsc.load_gather(Rl, [tv * H + h]))
                    al = jnp.maximum(al, al * F32(0.2))
                    ex = jnp.exp(al)
                    exb[pl.ds(g * 16, 16)] = ex
                    plsc.store_scatter(rowbuf,
                                       [ev, jnp.full((16,), OUT, I32)], ex)

                def _mul(g, _):
                    ev = exb[pl.ds(g * 16, 16)]
                    for i in range(16):
                        bi = ev.at[jnp.full((16,), i, I32)].get(
                            mode="promise_in_bounds")
                        e = g * 16 + i
                        for q in range(OUT // 16):
                            rowbuf[e, pl.ds(q * 16, 16)] = (
                                rowbuf[e, pl.ds(q * 16, 16)] * bi)
                    return 0
                lax.fori_loop(0, _G, _mul, 0)

            def _mkidx(ebuf, idxb):
                for g in range(_G):
                    sv = ebuf[0, pl.ds(g * 16, 16)]
                    idxb[0, pl.ds(g * 16, 16)] = sv + hN

            tbase = sid * (EC // _CH)

            def _fetch(cid0, ebuf, idxb, rowbuf, dbuf, esem, gsem, dsem):
                pltpu.async_copy(epack_hbm.at[cid0], ebuf, esem).wait()
                _mkidx(ebuf, idxb)
                pltpu.async_copy(ht_hbm.at[idxb.at[0]], rowbuf, gsem)
                pltpu.async_copy(tdst_hbm.at[ebuf.at[1]], dbuf, dsem)

            _fetch(tbase, ebuf0, idxb0, rowbuf0, dbuf0, esem0, gsem0, dsem0)
            _fetch(tbase + 1, ebuf1, idxb1, rowbuf1, dbuf1, esem1, gsem1,
                   dsem1)

            def _pair(j, _):
                pltpu.make_async_copy(ht_hbm.at[idxb0.at[0]], rowbuf0,
                                      gsem0).wait()
                pltpu.make_async_copy(tdst_hbm.at[ebuf0.at[1]], dbuf0,
                                      dsem0).wait()
                _proc(ebuf0, idxb0, exb0, rowbuf0, dbuf0)
                s0 = pltpu.async_copy(rowbuf0, acc_sh.at[ebuf0.at[1]], ssem0,
                                      add=True)
                pltpu.make_async_copy(ht_hbm.at[idxb1.at[0]], rowbuf1,
                                      gsem1).wait()
                pltpu.make_async_copy(tdst_hbm.at[ebuf1.at[1]], dbuf1,
                                      dsem1).wait()
                _proc(ebuf1, idxb1, exb1, rowbuf1, dbuf1)
                s1 = pltpu.async_copy(rowbuf1, acc_sh.at[ebuf1.at[1]], ssem1,
                                      add=True)
                nc = jnp.minimum(2 * j + 2, NCH - 2)
                s0.wait()
                _fetch(tbase + nc, ebuf0, idxb0, rowbuf0, dbuf0, esem0,
                       gsem0, dsem0)
                s1.wait()
                _fetch(tbase + nc + 1, ebuf1, idxb1, rowbuf1, dbuf1, esem1,
                       gsem1, dsem1)
                return 0

            lax.fori_loop(0, NCH // 2, _pair, 0)

            pltpu.make_async_copy(ht_hbm.at[idxb0.at[0]], rowbuf0,
                                  gsem0).wait()
            pltpu.make_async_copy(tdst_hbm.at[ebuf0.at[1]], dbuf0,
                                  dsem0).wait()
            pltpu.make_async_copy(ht_hbm.at[idxb1.at[0]], rowbuf1,
                                  gsem1).wait()
            pltpu.make_async_copy(tdst_hbm.at[ebuf1.at[1]], dbuf1,
                                  dsem1).wait()

            plsc.subcore_barrier()

            pltpu.sync_copy(acc_sh.at[pl.ds(sid * SR, SR)],
                            msum_hbm.at[pl.ds(hNP + sid * SR, SR)])

    return sc_kernel


def _final_body(msum_ref, bias_ref, out_ref):
    m = msum_ref[..., 0:128]
    d = msum_ref[..., 128:129]
    s = jnp.sum(m / (d + F32(1e-16)), axis=0) * F32(0.25)
    out_ref[...] = s + bias_ref[...]


def _make_final(N, NP, OUT, H, BN):
    return pl.pallas_call(
        _final_body,
        grid=(N // BN,),
        in_specs=[
            pl.BlockSpec((H, BN, _WD), lambda j: (0, j, 0)),
            pl.BlockSpec((1, OUT), lambda j: (0, 0)),
        ],
        out_specs=pl.BlockSpec((BN, OUT), lambda j: (j, 0)),
        out_shape=jax.ShapeDtypeStruct((N, OUT), F32),
    )


def kernel(x, edge_index, edge_type, rel_emb, W, W_edge, att_src, att_dst,
           att_edge, bias):
    N, IN = x.shape
    H, OUT = att_src.shape
    R = rel_emb.shape[0]
    E = edge_type.shape[0]
    NP = (N + _TILES * 16 - 1) // (_TILES * 16) * (_TILES * 16)
    assert E % (_TILES * _CH) == 0 and (NP // _TILES) % 16 == 0

    epack = jnp.stack([edge_index[0].reshape(E // _CH, _CH),
                       edge_index[1].reshape(E // _CH, _CH),
                       edge_type.reshape(E // _CH, _CH)], axis=1)

    ht, tdst, relt = _make_prep(N, IN, OUT, H, R, 400)(
        x, W, W_edge, rel_emb, att_src, att_dst, att_edge)

    msum = _make_sc(N, NP, OUT, H, R, E)(
        epack, ht.reshape(H * N, _WD), tdst, relt.reshape(R * H))

    out = _make_final(N, NP, OUT, H, 400)(
        msum.reshape(H, NP, _WD), bias.reshape(1, OUT))
    return out
